# baseline (speedup 1.0000x reference)
import functools
import jax
import jax.numpy as jnp
from jax import lax
from jax.experimental import pallas as pl
from jax.experimental.pallas import tpu as pltpu
from jax.experimental.pallas import tpu_sc as plsc

N = 10000
E = 320000
DIN = 128
DH = 256
DH2 = 128
NG = 64
NS = 16
NC = 2
K = 125
BQ = 16
R = 1000
GRID = N // R

_mesh = plsc.VectorSubcoreMesh(core_axis_name="c", subcore_axis_name="s")

_F32 = jnp.float32
_HIGH = lax.Precision.HIGHEST


_DEG_NCH = E // (NC * NS) // K


@functools.partial(
    pl.kernel,
    out_type=jax.ShapeDtypeStruct((NC, NS, N // NS, 16), _F32),
    mesh=_mesh,
    scratch_types=[
        pltpu.VMEM((_DEG_NCH, K), jnp.int32),
        pltpu.VMEM((K, 16), _F32),
        pltpu.VMEM_SHARED((N, 16), _F32),
    ],
)
def _deg_kernel(didx_hbm, out_hbm, idxs_v, rows, acc_sh):
    c = lax.axis_index("c")
    s = lax.axis_index("s")
    w = c * NS + s
    pltpu.sync_copy(didx_hbm.at[w], idxs_v)
    zero = jnp.zeros((16,), _F32)

    def z(i, carry):
        rows[i, :] = zero
        return carry

    lax.fori_loop(0, K, z, 0)
    rpt = N // NS

    def zc(q, carry):
        pltpu.sync_copy(rows, acc_sh.at[pl.ds(s * rpt + q * K, K)])
        return carry

    lax.fori_loop(0, rpt // K, zc, 0)
    e1 = jnp.where(lax.iota(jnp.int32, 16) == 0, 1.0, 0.0)

    def f(i, carry):
        rows[i, :] = e1
        return carry

    lax.fori_loop(0, K, f, 0)
    plsc.subcore_barrier()

    def step(j, carry):
        pltpu.sync_copy(rows, acc_sh.at[idxs_v.at[j]], add=True)
        return carry

    lax.fori_loop(0, _DEG_NCH, step, 0)
    plsc.subcore_barrier()
    pltpu.sync_copy(acc_sh.at[pl.ds(s * rpt, rpt)], out_hbm.at[c, s])


def _make_agg(nch, per_core_gidx):
    def body(tbl_hbm, gidx_hbm, didx_hbm, out_hbm, idxg_v, idxs_v, rows0,
             rows1, acc_sh, semg0, semg1, sems0, sems1):
        c = lax.axis_index("c")
        s = lax.axis_index("s")
        w = c * NS + s

        zero = jnp.zeros((16,), _F32)

        def z(i, carry):
            rows0[i // 8, pl.ds((i % 8) * 16, 16)] = zero
            return carry

        lax.fori_loop(0, K * 8, z, 0)
        rpt = N // NS

        def zc(q, carry):
            pltpu.sync_copy(rows0, acc_sh.at[pl.ds(s * rpt + q * K, K)])
            return carry

        lax.fori_loop(0, rpt // K, zc, 0)
        plsc.subcore_barrier()

        def block(b, carry):
            if per_core_gidx:
                pltpu.sync_copy(gidx_hbm.at[c, s, pl.ds(b * BQ, BQ)], idxg_v)
                pltpu.sync_copy(didx_hbm.at[s, pl.ds(b * BQ, BQ)], idxs_v)
            else:
                pltpu.sync_copy(gidx_hbm.at[w, pl.ds(b * BQ, BQ)], idxg_v)
                pltpu.sync_copy(didx_hbm.at[w, pl.ds(b * BQ, BQ)], idxs_v)
            pltpu.async_copy(tbl_hbm.at[idxg_v.at[0]], rows0, semg0)

            def step2(jj, carry2):
                j0 = jj * 2
                pltpu.async_copy(tbl_hbm.at[idxg_v.at[j0 + 1]], rows1, semg1)
                pltpu.make_async_copy(tbl_hbm.at[idxg_v.at[j0]], rows0,
                                      semg0).wait()
                pltpu.async_copy(rows0, acc_sh.at[idxs_v.at[j0]], sems0,
                                 add=True)
                pltpu.make_async_copy(tbl_hbm.at[idxg_v.at[j0 + 1]], rows1,
                                      semg1).wait()
                pltpu.async_copy(rows1, acc_sh.at[idxs_v.at[j0 + 1]], sems1,
                                 add=True)
                pltpu.make_async_copy(rows0, acc_sh.at[idxs_v.at[j0]],
                                      sems0).wait()

                @pl.when(jj + 1 < BQ // 2)
                def _():
                    pltpu.async_copy(tbl_hbm.at[idxg_v.at[j0 + 2]], rows0,
                                     semg0)

                pltpu.make_async_copy(rows1, acc_sh.at[idxs_v.at[j0 + 1]],
                                      sems1).wait()
                return carry2

            lax.fori_loop(0, BQ // 2, step2, 0)
            return carry

        lax.fori_loop(0, nch // BQ, block, 0)
        plsc.subcore_barrier()
        pltpu.sync_copy(acc_sh.at[pl.ds(s * rpt, rpt)], out_hbm.at[c, s])

    return functools.partial(
        pl.kernel,
        out_type=jax.ShapeDtypeStruct((NC, NS, N // NS, 128), _F32),
        mesh=_mesh,
        scratch_types=[
            pltpu.VMEM((BQ, K), jnp.int32),
            pltpu.VMEM((BQ, K), jnp.int32),
            pltpu.VMEM((K, 128), _F32),
            pltpu.VMEM((K, 128), _F32),
            pltpu.VMEM_SHARED((N, 128), _F32),
            pltpu.SemaphoreType.DMA,
            pltpu.SemaphoreType.DMA,
            pltpu.SemaphoreType.DMA,
            pltpu.SemaphoreType.DMA,
        ],
    )(body)


_NCH1 = E // NS // K
_NCH2 = E // (NC * NS) // K
_agg1_kernel = _make_agg(_NCH1, True)
_agg2_kernel = _make_agg(_NCH2, False)


def _dinv_block(degp_ref):
    i = pl.program_id(0)
    blk = degp_ref[:, pl.ds(i * R, R), :]
    deg = jnp.sum(jnp.sum(blk, axis=0), axis=-1)
    return lax.rsqrt(deg + 1.0)


def _mm1_body(degp_ref, x_ref, w_ref, out_ref):
    dinv = _dinv_block(degp_ref)
    h = jnp.dot(x_ref[...], w_ref[...], preferred_element_type=_F32,
                precision=_HIGH)
    out_ref[...] = h * dinv[:, None]


def _mm1(deg_part, x, Wc1):
    return pl.pallas_call(
        _mm1_body,
        grid=(GRID,),
        in_specs=[
            pl.BlockSpec((NC, N, 16), lambda i: (0, 0, 0)),
            pl.BlockSpec((R, DIN), lambda i: (i, 0)),
            pl.BlockSpec((DIN, DH), lambda i: (0, 0)),
        ],
        out_specs=pl.BlockSpec((R, DH), lambda i: (i, 0)),
        out_shape=jax.ShapeDtypeStruct((N, DH), _F32),
    )(deg_part, x, Wc1)


def _mm2_body(degp_ref, alo_ref, ahi_ref, hs_ref, b_ref, w_ref, out_ref):
    dinv = _dinv_block(degp_ref)[:, None]
    a = jnp.concatenate([alo_ref[...], ahi_ref[...]], axis=1) + hs_ref[...]
    h1 = jnp.maximum(a * dinv + b_ref[...], 0.0)
    h2 = jnp.dot(h1, w_ref[...], preferred_element_type=_F32, precision=_HIGH)
    out_ref[...] = h2 * dinv


def _mm2(deg_part, alo, ahi, hs, bc1, Wc2):
    return pl.pallas_call(
        _mm2_body,
        grid=(GRID,),
        in_specs=[
            pl.BlockSpec((NC, N, 16), lambda i: (0, 0, 0)),
            pl.BlockSpec((R, DH // 2), lambda i: (i, 0)),
            pl.BlockSpec((R, DH // 2), lambda i: (i, 0)),
            pl.BlockSpec((R, DH), lambda i: (i, 0)),
            pl.BlockSpec((1, DH), lambda i: (0, 0)),
            pl.BlockSpec((DH, DH2), lambda i: (0, 0)),
        ],
        out_specs=pl.BlockSpec((R, DH2), lambda i: (i, 0)),
        out_shape=jax.ShapeDtypeStruct((N, DH2), _F32),
    )(deg_part, alo, ahi, hs, bc1, Wc2)


def _pool_body(degp_ref, plo_ref, phi_ref, hs2_ref, b_ref, batch_ref, w1_ref,
               b1_ref, w2_ref, b2_ref, out_ref, acc):
    i = pl.program_id(0)
    dinv = _dinv_block(degp_ref)[:, None]
    y = (plo_ref[...] + phi_ref[...] + hs2_ref[...]) * dinv + b_ref[...]
    seg = batch_ref[0, 0, :]
    oh = (seg[:, None] == lax.broadcasted_iota(jnp.int32, (R, NG), 1)
          ).astype(_F32)
    yext = jnp.concatenate([y, jnp.ones((R, 1), _F32)], axis=1)
    part = lax.dot_general(oh, yext, (((0,), (0,)), ((), ())),
                           preferred_element_type=_F32, precision=_HIGH)

    @pl.when(i == 0)
    def _():
        acc[...] = part

    @pl.when(i > 0)
    def _():
        acc[...] += part

    @pl.when(i == pl.num_programs(0) - 1)
    def _():
        sums = acc[:, :DH2]
        cnt = jnp.maximum(acc[:, DH2:DH2 + 1], 1.0)
        g = jnp.maximum(sums / cnt, 0.0)
        z1 = jnp.maximum(
            jnp.dot(g, w1_ref[...], preferred_element_type=_F32,
                    precision=_HIGH) + b1_ref[...], 0.0)
        z2 = jnp.maximum(
            jnp.dot(z1, w2_ref[...], preferred_element_type=_F32,
                    precision=_HIGH) + b2_ref[...], 0.0)
        out_ref[...] = z2


def _pool(deg_part, plo, phi, hs2, bc2, batch3, W1, b1, W2, b2):
    return pl.pallas_call(
        _pool_body,
        grid=(GRID,),
        in_specs=[
            pl.BlockSpec((NC, N, 16), lambda i: (0, 0, 0)),
            pl.BlockSpec((R, DH2), lambda i: (i, 0)),
            pl.BlockSpec((R, DH2), lambda i: (i, 0)),
            pl.BlockSpec((R, DH2), lambda i: (i, 0)),
            pl.BlockSpec((1, DH2), lambda i: (0, 0)),
            pl.BlockSpec((1, 1, R), lambda i: (i, 0, 0)),
            pl.BlockSpec((DH2, NG), lambda i: (0, 0)),
            pl.BlockSpec((1, NG), lambda i: (0, 0)),
            pl.BlockSpec((NG, 10), lambda i: (0, 0)),
            pl.BlockSpec((1, 10), lambda i: (0, 0)),
        ],
        out_specs=pl.BlockSpec((NG, 10), lambda i: (0, 0)),
        out_shape=jax.ShapeDtypeStruct((NG, 10), _F32),
        scratch_shapes=[pltpu.VMEM((NG, DH2 + 1), _F32)],
    )(deg_part, plo, phi, hs2, bc2, batch3, W1, b1, W2, b2)


def kernel(x, edge_index, batch, Wc1, bc1, Wc2, bc2, W1, b1, W2, b2):
    src = edge_index[0]
    dst = edge_index[1]

    didx2 = dst.reshape(NC * NS, _NCH2, K)
    deg_part = _deg_kernel(didx2).reshape(NC, N, 16)

    hs = _mm1(deg_part, x, Wc1)
    hsv = hs.reshape(2 * N, DH // 2)

    gidx1 = (2 * src[None, :] + jnp.arange(NC, dtype=jnp.int32)[:, None]
             ).reshape(NC, NS, _NCH1, K)
    didx1 = dst.reshape(NS, _NCH1, K)
    agg1 = _agg1_kernel(hsv, gidx1, didx1).reshape(NC, N, DH // 2)

    hs2 = _mm2(deg_part, agg1[0], agg1[1], hs, bc1.reshape(1, DH), Wc2)

    gidx2 = src.reshape(NC * NS, _NCH2, K)
    p = _agg2_kernel(hs2, gidx2, didx2).reshape(NC, N, DH2)

    return _pool(deg_part, p[0], p[1], hs2, bc2.reshape(1, DH2),
                 batch.reshape(GRID, 1, R), W1, b1.reshape(1, NG), W2,
                 b2.reshape(1, 10))


# speedup vs baseline: 1.1914x; 1.1914x over previous
import functools
import jax
import jax.numpy as jnp
from jax import lax
from jax.experimental import pallas as pl
from jax.experimental.pallas import tpu as pltpu
from jax.experimental.pallas import tpu_sc as plsc

N = 10000
E = 320000
DIN = 128
DH = 256
DH2 = 128
NG = 64
NS = 16
NC = 2
K = 125
BQ = 16
R = 1000
GRID = N // R

_mesh = plsc.VectorSubcoreMesh(core_axis_name="c", subcore_axis_name="s")

_F32 = jnp.float32
_HIGH = lax.Precision.HIGHEST


_DEG_NCH = E // (NC * NS) // K


@functools.partial(
    pl.kernel,
    out_type=jax.ShapeDtypeStruct((NC, NS, N // NS, 16), _F32),
    mesh=_mesh,
    scratch_types=[
        pltpu.VMEM((_DEG_NCH, K), jnp.int32),
        pltpu.VMEM((K, 16), _F32),
        pltpu.VMEM_SHARED((N, 16), _F32),
    ],
)
def _deg_kernel(didx_hbm, out_hbm, idxs_v, rows, acc_sh):
    c = lax.axis_index("c")
    s = lax.axis_index("s")
    w = c * NS + s
    pltpu.sync_copy(didx_hbm.at[w], idxs_v)
    zero = jnp.zeros((16,), _F32)

    def z(i, carry):
        rows[i, :] = zero
        return carry

    lax.fori_loop(0, K, z, 0)
    rpt = N // NS

    def zc(q, carry):
        pltpu.sync_copy(rows, acc_sh.at[pl.ds(s * rpt + q * K, K)])
        return carry

    lax.fori_loop(0, rpt // K, zc, 0)
    e1 = jnp.where(lax.iota(jnp.int32, 16) == 0, 1.0, 0.0)

    def f(i, carry):
        rows[i, :] = e1
        return carry

    lax.fori_loop(0, K, f, 0)
    plsc.subcore_barrier()

    def step(j, carry):
        pltpu.sync_copy(rows, acc_sh.at[idxs_v.at[j]], add=True)
        return carry

    lax.fori_loop(0, _DEG_NCH, step, 0)
    plsc.subcore_barrier()
    pltpu.sync_copy(acc_sh.at[pl.ds(s * rpt, rpt)], out_hbm.at[c, s])


def _make_agg(nch, per_core_gidx):
    def body(tbl_hbm, gidx_hbm, didx_hbm, out_hbm, idxg_v, idxs_v, rows0,
             rows1, acc_sh, semg0, semg1, sems0, sems1):
        c = lax.axis_index("c")
        s = lax.axis_index("s")
        w = c * NS + s

        zero = jnp.zeros((16,), _F32)

        def z(i, carry):
            rows0[i // 8, pl.ds((i % 8) * 16, 16)] = zero
            return carry

        lax.fori_loop(0, K * 8, z, 0)
        rpt = N // NS

        def zc(q, carry):
            pltpu.sync_copy(rows0, acc_sh.at[pl.ds(s * rpt + q * K, K)])
            return carry

        lax.fori_loop(0, rpt // K, zc, 0)
        plsc.subcore_barrier()

        def block(b, carry):
            if per_core_gidx:
                pltpu.sync_copy(gidx_hbm.at[c, s, pl.ds(b * BQ, BQ)], idxg_v)
                pltpu.sync_copy(didx_hbm.at[s, pl.ds(b * BQ, BQ)], idxs_v)
            else:
                pltpu.sync_copy(gidx_hbm.at[w, pl.ds(b * BQ, BQ)], idxg_v)
                pltpu.sync_copy(didx_hbm.at[w, pl.ds(b * BQ, BQ)], idxs_v)
            pltpu.async_copy(tbl_hbm.at[idxg_v.at[0]], rows0, semg0)

            def step2(jj, carry2):
                j0 = jj * 2
                cp1 = pltpu.async_copy(tbl_hbm.at[idxg_v.at[j0 + 1]], rows1,
                                       semg1)
                pltpu.make_async_copy(tbl_hbm.at[idxg_v.at[j0]], rows0,
                                      semg0).wait()
                pltpu.sync_copy(rows0, acc_sh.at[idxs_v.at[j0]], add=True)

                @pl.when(jj + 1 < BQ // 2)
                def _():
                    pltpu.async_copy(tbl_hbm.at[idxg_v.at[j0 + 2]], rows0,
                                     semg0)

                cp1.wait()
                pltpu.sync_copy(rows1, acc_sh.at[idxs_v.at[j0 + 1]], add=True)
                return carry2

            lax.fori_loop(0, BQ // 2, step2, 0)
            return carry

        lax.fori_loop(0, nch // BQ, block, 0)
        plsc.subcore_barrier()
        pltpu.sync_copy(acc_sh.at[pl.ds(s * rpt, rpt)], out_hbm.at[c, s])

    return functools.partial(
        pl.kernel,
        out_type=jax.ShapeDtypeStruct((NC, NS, N // NS, 128), _F32),
        mesh=_mesh,
        scratch_types=[
            pltpu.VMEM((BQ, K), jnp.int32),
            pltpu.VMEM((BQ, K), jnp.int32),
            pltpu.VMEM((K, 128), _F32),
            pltpu.VMEM((K, 128), _F32),
            pltpu.VMEM_SHARED((N, 128), _F32),
            pltpu.SemaphoreType.DMA,
            pltpu.SemaphoreType.DMA,
            pltpu.SemaphoreType.DMA,
            pltpu.SemaphoreType.DMA,
        ],
    )(body)


_NCH1 = E // NS // K
_NCH2 = E // (NC * NS) // K
_agg1_kernel = _make_agg(_NCH1, True)
_agg2_kernel = _make_agg(_NCH2, False)


def _dinv_block(degp_ref):
    i = pl.program_id(0)
    blk = degp_ref[:, pl.ds(i * R, R), :]
    deg = jnp.sum(jnp.sum(blk, axis=0), axis=-1)
    return lax.rsqrt(deg + 1.0)


def _mm1_body(degp_ref, x_ref, w_ref, out_ref):
    dinv = _dinv_block(degp_ref)
    h = jnp.dot(x_ref[...], w_ref[...], preferred_element_type=_F32,
                precision=_HIGH)
    out_ref[...] = h * dinv[:, None]


def _mm1(deg_part, x, Wc1):
    return pl.pallas_call(
        _mm1_body,
        grid=(GRID,),
        in_specs=[
            pl.BlockSpec((NC, N, 16), lambda i: (0, 0, 0)),
            pl.BlockSpec((R, DIN), lambda i: (i, 0)),
            pl.BlockSpec((DIN, DH), lambda i: (0, 0)),
        ],
        out_specs=pl.BlockSpec((R, DH), lambda i: (i, 0)),
        out_shape=jax.ShapeDtypeStruct((N, DH), _F32),
    )(deg_part, x, Wc1)


def _mm2_body(degp_ref, alo_ref, ahi_ref, hs_ref, b_ref, w_ref, out_ref):
    dinv = _dinv_block(degp_ref)[:, None]
    a = jnp.concatenate([alo_ref[...], ahi_ref[...]], axis=1) + hs_ref[...]
    h1 = jnp.maximum(a * dinv + b_ref[...], 0.0)
    h2 = jnp.dot(h1, w_ref[...], preferred_element_type=_F32, precision=_HIGH)
    out_ref[...] = h2 * dinv


def _mm2(deg_part, alo, ahi, hs, bc1, Wc2):
    return pl.pallas_call(
        _mm2_body,
        grid=(GRID,),
        in_specs=[
            pl.BlockSpec((NC, N, 16), lambda i: (0, 0, 0)),
            pl.BlockSpec((R, DH // 2), lambda i: (i, 0)),
            pl.BlockSpec((R, DH // 2), lambda i: (i, 0)),
            pl.BlockSpec((R, DH), lambda i: (i, 0)),
            pl.BlockSpec((1, DH), lambda i: (0, 0)),
            pl.BlockSpec((DH, DH2), lambda i: (0, 0)),
        ],
        out_specs=pl.BlockSpec((R, DH2), lambda i: (i, 0)),
        out_shape=jax.ShapeDtypeStruct((N, DH2), _F32),
    )(deg_part, alo, ahi, hs, bc1, Wc2)


def _pool_body(degp_ref, plo_ref, phi_ref, hs2_ref, b_ref, batch_ref, w1_ref,
               b1_ref, w2_ref, b2_ref, out_ref, acc):
    i = pl.program_id(0)
    dinv = _dinv_block(degp_ref)[:, None]
    y = (plo_ref[...] + phi_ref[...] + hs2_ref[...]) * dinv + b_ref[...]
    seg = batch_ref[0, 0, :]
    oh = (seg[:, None] == lax.broadcasted_iota(jnp.int32, (R, NG), 1)
          ).astype(_F32)
    yext = jnp.concatenate([y, jnp.ones((R, 1), _F32)], axis=1)
    part = lax.dot_general(oh, yext, (((0,), (0,)), ((), ())),
                           preferred_element_type=_F32, precision=_HIGH)

    @pl.when(i == 0)
    def _():
        acc[...] = part

    @pl.when(i > 0)
    def _():
        acc[...] += part

    @pl.when(i == pl.num_programs(0) - 1)
    def _():
        sums = acc[:, :DH2]
        cnt = jnp.maximum(acc[:, DH2:DH2 + 1], 1.0)
        g = jnp.maximum(sums / cnt, 0.0)
        z1 = jnp.maximum(
            jnp.dot(g, w1_ref[...], preferred_element_type=_F32,
                    precision=_HIGH) + b1_ref[...], 0.0)
        z2 = jnp.maximum(
            jnp.dot(z1, w2_ref[...], preferred_element_type=_F32,
                    precision=_HIGH) + b2_ref[...], 0.0)
        out_ref[...] = z2


def _pool(deg_part, plo, phi, hs2, bc2, batch3, W1, b1, W2, b2):
    return pl.pallas_call(
        _pool_body,
        grid=(GRID,),
        in_specs=[
            pl.BlockSpec((NC, N, 16), lambda i: (0, 0, 0)),
            pl.BlockSpec((R, DH2), lambda i: (i, 0)),
            pl.BlockSpec((R, DH2), lambda i: (i, 0)),
            pl.BlockSpec((R, DH2), lambda i: (i, 0)),
            pl.BlockSpec((1, DH2), lambda i: (0, 0)),
            pl.BlockSpec((1, 1, R), lambda i: (i, 0, 0)),
            pl.BlockSpec((DH2, NG), lambda i: (0, 0)),
            pl.BlockSpec((1, NG), lambda i: (0, 0)),
            pl.BlockSpec((NG, 10), lambda i: (0, 0)),
            pl.BlockSpec((1, 10), lambda i: (0, 0)),
        ],
        out_specs=pl.BlockSpec((NG, 10), lambda i: (0, 0)),
        out_shape=jax.ShapeDtypeStruct((NG, 10), _F32),
        scratch_shapes=[pltpu.VMEM((NG, DH2 + 1), _F32)],
    )(deg_part, plo, phi, hs2, bc2, batch3, W1, b1, W2, b2)


def kernel(x, edge_index, batch, Wc1, bc1, Wc2, bc2, W1, b1, W2, b2):
    src = edge_index[0]
    dst = edge_index[1]

    didx2 = dst.reshape(NC * NS, _NCH2, K)
    deg_part = _deg_kernel(didx2).reshape(NC, N, 16)

    hs = _mm1(deg_part, x, Wc1)
    hsv = hs.reshape(2 * N, DH // 2)

    gidx1 = (2 * src[None, :] + jnp.arange(NC, dtype=jnp.int32)[:, None]
             ).reshape(NC, NS, _NCH1, K)
    didx1 = dst.reshape(NS, _NCH1, K)
    agg1 = _agg1_kernel(hsv, gidx1, didx1).reshape(NC, N, DH // 2)

    hs2 = _mm2(deg_part, agg1[0], agg1[1], hs, bc1.reshape(1, DH), Wc2)

    gidx2 = src.reshape(NC * NS, _NCH2, K)
    p = _agg2_kernel(hs2, gidx2, didx2).reshape(NC, N, DH2)

    return _pool(deg_part, p[0], p[1], hs2, bc2.reshape(1, DH2),
                 batch.reshape(GRID, 1, R), W1, b1.reshape(1, NG), W2,
                 b2.reshape(1, 10))
